# baseline (speedup 1.0000x reference)
import functools

import jax
import jax.numpy as jnp
from jax import lax
from jax.experimental import pallas as pl
from jax.experimental.pallas import tpu as pltpu
from jax.experimental.pallas import tpu_sc as plsc

NODE_SIZE = 100000
PATH_SIZE = 64
EMBED_DIM = 128
NUM_ETYPES = 4
NEI = 16
BATCH = 1024

NC = 2
NS = 16
NW = NC * NS
BPW = BATCH // NW
ROWS = NUM_ETYPES * NEI


def _sc_body(nbr_hbm, sidx_hbm, eidx_hbm, pidx_hbm, estart_hbm, eend_hbm,
             epath_hbm, agg_hbm, eemb_hbm, praw_hbm,
             idx_v, ridx_v, nbr_v, sel_v, rows_v, out_v, misc_v, sem):
    wid = lax.axis_index("s") * NC + lax.axis_index("c")
    base = wid * BPW

    pltpu.sync_copy(sidx_hbm.at[pl.ds(base, BPW)], idx_v)
    for c in range(BPW // 16):
        sl = pl.ds(c * 16, 16)
        ridx_v[sl] = lax.shift_right_logical(idx_v[sl], 1)
    pltpu.async_copy(nbr_hbm.at[ridx_v], nbr_v, sem).wait()

    def body(j, carry):
        jj = jnp.full((16,), j, jnp.int32)
        sj = plsc.load_gather(idx_v, [jj])
        par = (sj & 1) == 1
        for c in range(ROWS // 16):
            a = nbr_v[j, pl.ds(c * 16, 16)]
            b = nbr_v[j, pl.ds((c + ROWS // 16) * 16, 16)]
            sel_v[pl.ds(c * 16, 16)] = jnp.where(par, b, a)
        pltpu.async_copy(estart_hbm.at[sel_v], rows_v, sem).wait()
        for e in range(NUM_ETYPES):
            for c in range(EMBED_DIM // 16):
                sl = pl.ds(c * 16, 16)
                vals = [rows_v[e * NEI + r, sl] for r in range(NEI)]
                while len(vals) > 1:
                    vals = [vals[i] + vals[i + 1] for i in range(0, len(vals), 2)]
                out_v[e, j, sl] = vals[0] * (1.0 / NEI)
        return carry

    lax.fori_loop(0, BPW, body, 0)
    for e in range(NUM_ETYPES):
        pltpu.sync_copy(out_v.at[e], agg_hbm.at[e, pl.ds(base, BPW)])

    pltpu.sync_copy(eidx_hbm.at[pl.ds(base, BPW)], idx_v)
    pltpu.async_copy(eend_hbm.at[idx_v], misc_v, sem).wait()
    pltpu.sync_copy(misc_v, eemb_hbm.at[pl.ds(base, BPW)])

    pltpu.sync_copy(pidx_hbm.at[pl.ds(base, BPW)], idx_v)
    pltpu.async_copy(epath_hbm.at[idx_v], misc_v, sem).wait()
    pltpu.sync_copy(misc_v, praw_hbm.at[pl.ds(base, BPW)])


_sc_gather = functools.partial(
    pl.kernel,
    out_type=(
        jax.ShapeDtypeStruct((NUM_ETYPES, BATCH, EMBED_DIM), jnp.float32),
        jax.ShapeDtypeStruct((BATCH, EMBED_DIM), jnp.float32),
        jax.ShapeDtypeStruct((BATCH, EMBED_DIM), jnp.float32),
    ),
    mesh=plsc.VectorSubcoreMesh(
        core_axis_name="c", subcore_axis_name="s", num_cores=NC,
        num_subcores=NS),
    compiler_params=pltpu.CompilerParams(needs_layout_passes=False),
    scratch_types=[
        pltpu.VMEM((BPW,), jnp.int32),
        pltpu.VMEM((BPW,), jnp.int32),
        pltpu.VMEM((BPW, 2 * ROWS), jnp.int32),
        pltpu.VMEM((ROWS,), jnp.int32),
        pltpu.VMEM((ROWS, EMBED_DIM), jnp.float32),
        pltpu.VMEM((NUM_ETYPES, BPW, EMBED_DIM), jnp.float32),
        pltpu.VMEM((BPW, EMBED_DIM), jnp.float32),
        pltpu.SemaphoreType.DMA,
    ],
)(_sc_body)


def _tc_body(agg_ref, eemb_ref, praw_ref, W1_ref, b1_ref, W2_ref, b2_ref,
             out_ref):
    f32 = jnp.float32
    hi = lax.Precision.HIGHEST
    W1 = W1_ref[...]
    b1 = b1_ref[...]
    acc = jnp.broadcast_to(b2_ref[...], (BATCH, EMBED_DIM))
    for e in range(NUM_ETYPES):
        h = lax.dot_general(agg_ref[e], W1, (((1,), (0,)), ((), ())),
                            precision=hi, preferred_element_type=f32) + b1
        W2e = W2_ref[pl.ds(e * EMBED_DIM, EMBED_DIM), :]
        acc = acc + lax.dot_general(h, W2e, (((1,), (0,)), ((), ())),
                                    precision=hi, preferred_element_type=f32)
    p = jax.nn.sigmoid(praw_ref[...])
    m = acc * eemb_ref[...] * p
    out_ref[...] = jax.nn.sigmoid(jnp.sum(m, axis=1, keepdims=True))


_tc_dense = pl.pallas_call(
    _tc_body,
    out_shape=jax.ShapeDtypeStruct((BATCH, 1), jnp.float32),
)


def kernel(neighbors, start_node, end_node, path, embeds_start, embeds_end,
           embeds_path, W1, b1, W2, b2):
    nbr_flat = neighbors.reshape(NODE_SIZE // 2, 2 * ROWS)
    agg, eemb, praw = _sc_gather(
        nbr_flat, start_node.astype(jnp.int32), end_node.astype(jnp.int32),
        path.astype(jnp.int32), embeds_start, embeds_end, embeds_path)
    out = _tc_dense(agg, eemb, praw, W1, b1.reshape(1, EMBED_DIM), W2,
                    b2.reshape(1, EMBED_DIM))
    return out.reshape(BATCH)


# speedup vs baseline: 1.2014x; 1.2014x over previous
import functools

import jax
import jax.numpy as jnp
from jax import lax
from jax.experimental import pallas as pl
from jax.experimental.pallas import tpu as pltpu
from jax.experimental.pallas import tpu_sc as plsc

NODE_SIZE = 100000
PATH_SIZE = 64
EMBED_DIM = 128
NUM_ETYPES = 4
NEI = 16
BATCH = 1024

NC = 2
NS = 16
NW = NC * NS
BPW = BATCH // NW
ROWS = NUM_ETYPES * NEI


def _sc_body(nbr_hbm, sidx_hbm, eidx_hbm, pidx_hbm, estart_hbm, eend_hbm,
             epath_hbm, agg_hbm, eemb_hbm, praw_hbm,
             idx_v, nbr_v, rows_v, out_v, misc_v, sem):
    wid = lax.axis_index("s") * NC + lax.axis_index("c")
    base = wid * BPW

    pltpu.sync_copy(sidx_hbm.at[pl.ds(base, BPW)], idx_v)

    def fetch_ids(cc, carry):
        vj = idx_v[pl.ds(cc * 16, 16)]
        cps = [
            pltpu.async_copy(nbr_hbm.at[vj[k]], nbr_v.at[cc * 16 + k], sem)
            for k in range(16)
        ]
        for cp in cps:
            cp.wait()
        return carry

    lax.fori_loop(0, BPW // 16, fetch_ids, 0)

    def body(j, carry):
        cps = [
            pltpu.async_copy(estart_hbm.at[nbr_v.at[j, e]], rows_v.at[e], sem)
            for e in range(NUM_ETYPES)
        ]
        for cp in cps:
            cp.wait()
        for e in range(NUM_ETYPES):
            for c in range(EMBED_DIM // 16):
                sl = pl.ds(c * 16, 16)
                vals = [rows_v[e, r, sl] for r in range(NEI)]
                while len(vals) > 1:
                    vals = [vals[i] + vals[i + 1]
                            for i in range(0, len(vals), 2)]
                out_v[e, j, sl] = vals[0] * (1.0 / NEI)
        return carry

    lax.fori_loop(0, BPW, body, 0)
    for e in range(NUM_ETYPES):
        pltpu.sync_copy(out_v.at[e], agg_hbm.at[e, pl.ds(base, BPW)])

    pltpu.sync_copy(eidx_hbm.at[pl.ds(base, BPW)], idx_v)
    pltpu.async_copy(eend_hbm.at[idx_v], misc_v, sem).wait()
    pltpu.sync_copy(misc_v, eemb_hbm.at[pl.ds(base, BPW)])

    pltpu.sync_copy(pidx_hbm.at[pl.ds(base, BPW)], idx_v)
    pltpu.async_copy(epath_hbm.at[idx_v], misc_v, sem).wait()
    pltpu.sync_copy(misc_v, praw_hbm.at[pl.ds(base, BPW)])


_sc_gather = functools.partial(
    pl.kernel,
    out_type=(
        jax.ShapeDtypeStruct((NUM_ETYPES, BATCH, EMBED_DIM), jnp.float32),
        jax.ShapeDtypeStruct((BATCH, EMBED_DIM), jnp.float32),
        jax.ShapeDtypeStruct((BATCH, EMBED_DIM), jnp.float32),
    ),
    mesh=plsc.VectorSubcoreMesh(
        core_axis_name="c", subcore_axis_name="s", num_cores=NC,
        num_subcores=NS),
    compiler_params=pltpu.CompilerParams(needs_layout_passes=False),
    scratch_types=[
        pltpu.VMEM((BPW,), jnp.int32),
        pltpu.VMEM((BPW, NUM_ETYPES, NEI), jnp.int32),
        pltpu.VMEM((NUM_ETYPES, NEI, EMBED_DIM), jnp.float32),
        pltpu.VMEM((NUM_ETYPES, BPW, EMBED_DIM), jnp.float32),
        pltpu.VMEM((BPW, EMBED_DIM), jnp.float32),
        pltpu.SemaphoreType.DMA,
    ],
)(_sc_body)


def _tc_body(agg_ref, eemb_ref, praw_ref, W1_ref, b1_ref, W2_ref, b2_ref,
             out_ref):
    f32 = jnp.float32
    hi = lax.Precision.HIGHEST
    W1 = W1_ref[...]
    b1 = b1_ref[...]
    acc = jnp.broadcast_to(b2_ref[...], (BATCH, EMBED_DIM))
    for e in range(NUM_ETYPES):
        h = lax.dot_general(agg_ref[e], W1, (((1,), (0,)), ((), ())),
                            precision=hi, preferred_element_type=f32) + b1
        W2e = W2_ref[pl.ds(e * EMBED_DIM, EMBED_DIM), :]
        acc = acc + lax.dot_general(h, W2e, (((1,), (0,)), ((), ())),
                                    precision=hi, preferred_element_type=f32)
    p = jax.nn.sigmoid(praw_ref[...])
    m = acc * eemb_ref[...] * p
    out_ref[...] = jax.nn.sigmoid(jnp.sum(m, axis=1, keepdims=True))


_tc_dense = pl.pallas_call(
    _tc_body,
    out_shape=jax.ShapeDtypeStruct((BATCH, 1), jnp.float32),
)


def kernel(neighbors, start_node, end_node, path, embeds_start, embeds_end,
           embeds_path, W1, b1, W2, b2):
    agg, eemb, praw = _sc_gather(
        neighbors, start_node.astype(jnp.int32), end_node.astype(jnp.int32),
        path.astype(jnp.int32), embeds_start, embeds_end, embeds_path)
    out = _tc_dense(agg, eemb, praw, W1, b1.reshape(1, EMBED_DIM), W2,
                    b2.reshape(1, EMBED_DIM))
    return out.reshape(BATCH)
